# Initial kernel scaffold; baseline (speedup 1.0000x reference)
import jax
import jax.numpy as jnp
from jax.experimental import pallas as pl


def kernel(ui, uv, ai, av, y, a_table, u_table, fc_w, fc_b):
    raise NotImplementedError("write your pallas kernel here")


# speedup vs baseline: 12.9898x; 12.9898x over previous
import jax
import jax.numpy as jnp
from jax import lax
from jax.experimental import pallas as pl
from jax.experimental.pallas import tpu as pltpu
from jax.experimental.pallas import tpu_sc as plsc

B = 4096
AJ = 182
UJ = 26
E = 16
NC, NS = 2, 16
NW = NC * NS
BPW = B // NW
CB = 16
NCHUNK = BPW // CB


def _fire_gathers(table_h, idx_v, rows_v, nrows, sem):
    cps = []
    for off in range(0, nrows, 128):
        ln = min(128, nrows - off)
        cps.append(pltpu.async_copy(
            table_h.at[idx_v.at[pl.ds(off, ln)]],
            rows_v.at[pl.ds(off, ln)], sem))
    return cps


def _segment_acc(rows_v, w_v, val_v, nj, acc):
    lane = lax.iota(jnp.int32, 16)
    rowstep = lane * nj
    cols = [jnp.full((16,), e, jnp.int32) for e in range(E)]

    def jbody(j, acc):
        rowidx = rowstep + j
        val16 = plsc.load_gather(val_v, [rowidx])
        wrow = w_v[j]
        wacc = [jnp.zeros((16,), jnp.float32) for _ in range(4)]
        for e in range(E):
            col16 = plsc.load_gather(rows_v, [rowidx, cols[e]])
            wacc[e % 4] = wacc[e % 4] + col16 * wrow[e]
        w01 = wacc[0] + wacc[1]
        w23 = wacc[2] + wacc[3]
        return acc + val16 * (w01 + w23)

    return lax.fori_loop(0, nj, jbody, acc, unroll=2)


def _sc_body(a_table_h, u_table_h, a_idx_h, u_idx_h, a_val_h, u_val_h,
             w_a_h, w_u_h, y_h, fcb_h,
             yp_out, sq_out,
             a_idx_v, u_idx_v, a_rows_v, u_rows_v, a_val_v, u_val_v,
             w_a_v, w_u_v, y_v, yp_v, stage_v,
             sem_g, sem_i, sem_v):
    c = lax.axis_index("c")
    s = lax.axis_index("s")
    wid = s * NC + c
    base = wid * BPW

    pltpu.sync_copy(w_a_h, w_a_v)
    pltpu.sync_copy(w_u_h, w_u_v)
    pltpu.sync_copy(y_h.at[pl.ds(base, BPW)], y_v)
    pltpu.sync_copy(fcb_h, stage_v)

    def fire_idx(c):
        p = c % 2
        cb = base + c * CB
        return [
            pltpu.async_copy(a_idx_h.at[pl.ds(cb * AJ, CB * AJ)],
                             a_idx_v.at[p], sem_i.at[p]),
            pltpu.async_copy(u_idx_h.at[pl.ds(cb * UJ, CB * UJ)],
                             u_idx_v.at[p], sem_i.at[p]),
        ]

    def fire_val(c):
        p = c % 2
        cb = base + c * CB
        return [
            pltpu.async_copy(a_val_h.at[pl.ds(cb * AJ, CB * AJ)],
                             a_val_v.at[p], sem_v.at[p]),
            pltpu.async_copy(u_val_h.at[pl.ds(cb * UJ, CB * UJ)],
                             u_val_v.at[p], sem_v.at[p]),
        ]

    def fire_rows(c):
        p = c % 2
        cps = _fire_gathers(a_table_h, a_idx_v.at[p], a_rows_v.at[p],
                            CB * AJ, sem_g.at[p])
        cps += _fire_gathers(u_table_h, u_idx_v.at[p], u_rows_v.at[p],
                             CB * UJ, sem_g.at[p])
        return cps

    pend_i = {0: fire_idx(0), 1: fire_idx(1)}
    pend_v = {0: fire_val(0), 1: fire_val(1)}
    for cp in pend_i.pop(0):
        cp.wait()
    pend_g = {0: fire_rows(0)}

    for chunk in range(NCHUNK):
        for cp in pend_g.pop(chunk):
            cp.wait()
        for cp in pend_v.pop(chunk):
            cp.wait()
        if chunk + 1 < NCHUNK:
            for cp in pend_i.pop(chunk + 1):
                cp.wait()
            pend_g[chunk + 1] = fire_rows(chunk + 1)
        if chunk + 2 < NCHUNK:
            pend_i[chunk + 2] = fire_idx(chunk + 2)

        p = chunk % 2
        acc = stage_v[...]
        acc = _segment_acc(a_rows_v.at[p], w_a_v, a_val_v.at[p], AJ, acc)
        acc = _segment_acc(u_rows_v.at[p], w_u_v, u_val_v.at[p], UJ, acc)
        yp_v[pl.ds(chunk * CB, 16)] = acc

        if chunk + 2 < NCHUNK:
            pend_v[chunk + 2] = fire_val(chunk + 2)

    sqa = jnp.zeros((16,), jnp.float32)
    for k in range(BPW // 16):
        d = yp_v[pl.ds(k * 16, 16)] - y_v[pl.ds(k * 16, 16)]
        sqa = sqa + d * d
    pltpu.sync_copy(yp_v, yp_out.at[pl.ds(base, BPW)])
    stage_v[...] = sqa
    pltpu.sync_copy(stage_v, sq_out.at[wid])


@jax.jit
def _run(a_table, u_table, a_idx_f, u_idx_f, a_val_f, u_val_f, w_a, w_u,
         y_f, fcb16):
    mesh = plsc.VectorSubcoreMesh(core_axis_name="c", subcore_axis_name="s",
                                  num_cores=NC, num_subcores=NS)
    f = pl.kernel(
        _sc_body,
        out_type=(jax.ShapeDtypeStruct((B,), jnp.float32),
                  jax.ShapeDtypeStruct((NW, 16), jnp.float32)),
        mesh=mesh,
        compiler_params=pltpu.CompilerParams(needs_layout_passes=False,
                                             use_tc_tiling_on_sc=False),
        scratch_types=[
            pltpu.VMEM((2, CB * AJ), jnp.int32),
            pltpu.VMEM((2, CB * UJ), jnp.int32),
            pltpu.VMEM((2, CB * AJ, E), jnp.float32),
            pltpu.VMEM((2, CB * UJ, E), jnp.float32),
            pltpu.VMEM((2, CB * AJ), jnp.float32),
            pltpu.VMEM((2, CB * UJ), jnp.float32),
            pltpu.VMEM((AJ, E), jnp.float32),
            pltpu.VMEM((UJ, E), jnp.float32),
            pltpu.VMEM((BPW,), jnp.float32),
            pltpu.VMEM((BPW,), jnp.float32),
            pltpu.VMEM((16,), jnp.float32),
            pltpu.SemaphoreType.DMA((2,)),
            pltpu.SemaphoreType.DMA((2,)),
            pltpu.SemaphoreType.DMA((2,)),
        ],
    )
    return f(a_table, u_table, a_idx_f, u_idx_f, a_val_f, u_val_f, w_a, w_u,
             y_f, fcb16)


def kernel(ui, uv, ai, av, y, a_table, u_table, fc_w, fc_b):
    a_idx_f = ai.reshape(-1).astype(jnp.int32)
    u_idx_f = ui.reshape(-1).astype(jnp.int32)
    a_val_f = av.reshape(-1)
    u_val_f = uv.reshape(-1)
    w_a = fc_w[:AJ * E].reshape(AJ, E)
    w_u = fc_w[AJ * E:].reshape(UJ, E)
    y_f = y.reshape(B)
    fcb16 = jnp.full((16,), fc_b[0], jnp.float32)

    yp, sq = _run(a_table, u_table, a_idx_f, u_idx_f, a_val_f, u_val_f,
                  w_a, w_u, y_f, fcb16)
    y_pred = yp.reshape(B, 1)
    loss = jnp.sum(sq) / B
    return (loss, y_pred)
